# Initial kernel scaffold; baseline (speedup 1.0000x reference)
import jax
import jax.numpy as jnp
from jax.experimental import pallas as pl


def kernel(k, v, scores, n_evict):
    raise NotImplementedError("write your pallas kernel here")


# speedup vs baseline: 2.2938x; 2.2938x over previous
import jax
import jax.numpy as jnp
from jax import lax
from jax.experimental import pallas as pl
from jax.experimental.pallas import tpu as pltpu
from jax.experimental.pallas import tpu_sc as plsc

B, H, N, D = 8, 16, 8192, 64
KEEP = 6144
EVICT = 2048

BI = 512


def _rank_body(srow_ref, scol_ref, out_ref):
    b_idx = pl.program_id(0)
    i_blk = pl.program_id(1)
    s_j = srow_ref[0, :, :]
    col = lax.broadcasted_iota(jnp.int32, (BI, B), 1)
    s_i = jnp.sum(jnp.where(col == b_idx, scol_ref[0], 0.0), axis=1,
                  keepdims=True)
    j_idx = lax.broadcasted_iota(jnp.int32, (BI, N), 1)
    i_idx = lax.broadcasted_iota(jnp.int32, (BI, N), 0) + i_blk * BI
    before = (s_j > s_i) | ((s_j == s_i) & (j_idx < i_idx))
    out_ref[0, 0, 0, :] = jnp.sum(before.astype(jnp.int32), axis=1)


def _ranks(scores, scores_t):
    out = pl.pallas_call(
        _rank_body,
        grid=(B, N // BI),
        in_specs=[
            pl.BlockSpec((1, 1, N), lambda b, i: (b, 0, 0)),
            pl.BlockSpec((1, BI, B), lambda b, i: (i, 0, 0)),
        ],
        out_specs=pl.BlockSpec((1, 1, 1, BI), lambda b, i: (b, i, 0, 0)),
        out_shape=jax.ShapeDtypeStruct((B, N // BI, 1, BI), jnp.int32),
    )(scores.reshape(B, 1, N), scores_t.reshape(N // BI, BI, B))
    return out.reshape(B, N)


def _invert_body(rank_ref, srow_ref, sidx_ref, ss_ref):
    r_blk = pl.program_id(1)
    rank_row = rank_ref[0, :, :]
    s_row = srow_ref[0, :, :]
    r_idx = lax.broadcasted_iota(jnp.int32, (BI, N), 0) + r_blk * BI
    j_idx = lax.broadcasted_iota(jnp.int32, (BI, N), 1)
    eq = rank_row == r_idx
    sidx_ref[0, 0, 0, :] = jnp.sum(jnp.where(eq, j_idx, 0), axis=1)
    ss_ref[0, 0, 0, :] = jnp.sum(jnp.where(eq, s_row, 0.0), axis=1)


def _invert(rank, scores):
    sidx, ss = pl.pallas_call(
        _invert_body,
        grid=(B, N // BI),
        in_specs=[
            pl.BlockSpec((1, 1, N), lambda b, i: (b, 0, 0)),
            pl.BlockSpec((1, 1, N), lambda b, i: (b, 0, 0)),
        ],
        out_specs=[
            pl.BlockSpec((1, 1, 1, BI), lambda b, i: (b, i, 0, 0)),
            pl.BlockSpec((1, 1, 1, BI), lambda b, i: (b, i, 0, 0)),
        ],
        out_shape=[
            jax.ShapeDtypeStruct((B, N // BI, 1, BI), jnp.int32),
            jax.ShapeDtypeStruct((B, N // BI, 1, BI), jnp.float32),
        ],
    )(rank.reshape(B, 1, N), scores.reshape(B, 1, N))
    return sidx.reshape(B, N), ss.reshape(B, N)


NC, NS = 2, 16
NW = NC * NS
TPW = (B * H) // NW
LANES = 16
W = 512
NWIN = N // W
KWIN = KEEP // W
EWIN = EVICT // W


def _sc_body(kf, vf, sidx_hbm,
             kk, kv, ek, ev,
             sidx_v, idxw_v, kbuf_v, vbuf_v, dsem):
    c = lax.axis_index("c")
    s = lax.axis_index("s")
    wid = s * NC + c
    b = wid // TPW

    pltpu.sync_copy(sidx_hbm.at[b], sidx_v)

    for t in range(TPW):
        bh = wid * TPW + t
        off = bh * N

        def do_win(wi, dstk, dstv, off=off):
            def mk(ci, c2):
                sl = pl.ds(ci * LANES, LANES)
                idxw_v[sl] = sidx_v[pl.ds(wi * W + ci * LANES, LANES)] + off
                return c2
            lax.fori_loop(0, W // LANES, mk, 0)
            pltpu.async_copy(kf.at[idxw_v], kbuf_v, dsem).wait()
            pltpu.sync_copy(kbuf_v, dstk)
            pltpu.async_copy(vf.at[idxw_v], vbuf_v, dsem).wait()
            pltpu.sync_copy(vbuf_v, dstv)

        def keep_win(wi, carry, bh=bh):
            do_win(wi, kk.at[bh, pl.ds(wi * W, W)], kv.at[bh, pl.ds(wi * W, W)])
            return carry

        lax.fori_loop(0, KWIN, keep_win, 0)

        def ev_win(wj, carry, bh=bh):
            do_win(KWIN + wj,
                   ek.at[bh, pl.ds(wj * W, W)], ev.at[bh, pl.ds(wj * W, W)])
            return carry

        lax.fori_loop(0, EWIN, ev_win, 0)


def _make_sc_gather():
    return pl.kernel(
        _sc_body,
        out_type=(
            jax.ShapeDtypeStruct((B * H, KEEP, D), jnp.float32),
            jax.ShapeDtypeStruct((B * H, KEEP, D), jnp.float32),
            jax.ShapeDtypeStruct((B * H, EVICT, D), jnp.float32),
            jax.ShapeDtypeStruct((B * H, EVICT, D), jnp.float32),
        ),
        mesh=plsc.VectorSubcoreMesh(
            core_axis_name="c", subcore_axis_name="s",
            num_cores=NC, num_subcores=NS),
        compiler_params=pltpu.CompilerParams(use_tc_tiling_on_sc=False),
        scratch_types=[
            pltpu.VMEM((N,), jnp.int32),
            pltpu.VMEM((W,), jnp.int32),
            pltpu.VMEM((W, D), jnp.float32),
            pltpu.VMEM((W, D), jnp.float32),
            pltpu.SemaphoreType.DMA,
        ],
    )


def kernel(k, v, scores, n_evict):
    del n_evict
    rank = _ranks(scores, scores.T)
    sidx, ss = _invert(rank, scores)
    kf = k.reshape(B * H * N, D)
    vf = v.reshape(B * H * N, D)
    kk, kv, ek, ev = _make_sc_gather()(kf, vf, sidx)
    return (
        kk.reshape(B, H, KEEP, D),
        kv.reshape(B, H, KEEP, D),
        ss[:, :KEEP],
        ek.reshape(B, H, EVICT, D),
        ev.reshape(B, H, EVICT, D),
        ss[:, KEEP:],
    )


# speedup vs baseline: 3.0168x; 1.3152x over previous
import jax
import jax.numpy as jnp
from jax import lax
from jax.experimental import pallas as pl
from jax.experimental.pallas import tpu as pltpu
from jax.experimental.pallas import tpu_sc as plsc

B, H, N, D = 8, 16, 8192, 64
KEEP = 6144
EVICT = 2048

def _sort_body(jt_ref, kt_ref, s_in_ref, ss_ref, si_ref):
    step = pl.program_id(0)

    @pl.when(step == 0)
    def _():
        ss_ref[...] = s_in_ref[...]
        si_ref[...] = lax.broadcasted_iota(jnp.int32, (B, N), 1)

    j = jt_ref[step]
    k = kt_ref[step]
    s = ss_ref[...]
    ix = si_ref[...]
    iota = lax.broadcasted_iota(jnp.int32, (B, N), 1)
    bitj = (iota & j) == 0
    sp = jnp.where(bitj, pltpu.roll(s, N - j, 1), pltpu.roll(s, j, 1))
    ip = jnp.where(bitj, pltpu.roll(ix, N - j, 1), pltpu.roll(ix, j, 1))
    up = (iota & k) == 0
    lt_peer = (sp > s) | ((sp == s) & (ip < ix))
    take = lt_peer == (bitj == up)
    ss_ref[...] = jnp.where(take, sp, s)
    si_ref[...] = jnp.where(take, ip, ix)


def _bitonic_steps():
    js, ks = [], []
    k = 2
    while k <= N:
        j = k // 2
        while j >= 1:
            js.append(j)
            ks.append(k)
            j //= 2
        k *= 2
    return js, ks


def _sort_scores(scores):
    js, ks = _bitonic_steps()
    jt = jnp.asarray(js, dtype=jnp.int32)
    kt = jnp.asarray(ks, dtype=jnp.int32)
    ss, si = pl.pallas_call(
        _sort_body,
        grid=(len(js),),
        in_specs=[
            pl.BlockSpec(memory_space=pltpu.SMEM),
            pl.BlockSpec(memory_space=pltpu.SMEM),
            pl.BlockSpec((B, N), lambda i: (0, 0)),
        ],
        out_specs=[
            pl.BlockSpec((B, N), lambda i: (0, 0)),
            pl.BlockSpec((B, N), lambda i: (0, 0)),
        ],
        out_shape=[
            jax.ShapeDtypeStruct((B, N), jnp.float32),
            jax.ShapeDtypeStruct((B, N), jnp.int32),
        ],
    )(jt, kt, scores)
    return ss, si


NC, NS = 2, 16
NW = NC * NS
TPW = (B * H) // NW
LANES = 16
W = 512
NWIN = N // W
KWIN = KEEP // W
EWIN = EVICT // W


def _sc_body(kf, vf, sidx_hbm,
             kk, kv, ek, ev,
             sidx_v, idxw_v, kbuf_v, vbuf_v, dsem):
    c = lax.axis_index("c")
    s = lax.axis_index("s")
    wid = s * NC + c
    b = wid // TPW

    pltpu.sync_copy(sidx_hbm.at[b], sidx_v)

    for t in range(TPW):
        bh = wid * TPW + t
        off = bh * N

        def do_win(wi, dstk, dstv, off=off):
            def mk(ci, c2):
                sl = pl.ds(ci * LANES, LANES)
                idxw_v[sl] = sidx_v[pl.ds(wi * W + ci * LANES, LANES)] + off
                return c2
            lax.fori_loop(0, W // LANES, mk, 0)
            pltpu.async_copy(kf.at[idxw_v], kbuf_v, dsem).wait()
            pltpu.sync_copy(kbuf_v, dstk)
            pltpu.async_copy(vf.at[idxw_v], vbuf_v, dsem).wait()
            pltpu.sync_copy(vbuf_v, dstv)

        def keep_win(wi, carry, bh=bh):
            do_win(wi, kk.at[bh, pl.ds(wi * W, W)], kv.at[bh, pl.ds(wi * W, W)])
            return carry

        lax.fori_loop(0, KWIN, keep_win, 0)

        def ev_win(wj, carry, bh=bh):
            do_win(KWIN + wj,
                   ek.at[bh, pl.ds(wj * W, W)], ev.at[bh, pl.ds(wj * W, W)])
            return carry

        lax.fori_loop(0, EWIN, ev_win, 0)


def _make_sc_gather():
    return pl.kernel(
        _sc_body,
        out_type=(
            jax.ShapeDtypeStruct((B * H, KEEP, D), jnp.float32),
            jax.ShapeDtypeStruct((B * H, KEEP, D), jnp.float32),
            jax.ShapeDtypeStruct((B * H, EVICT, D), jnp.float32),
            jax.ShapeDtypeStruct((B * H, EVICT, D), jnp.float32),
        ),
        mesh=plsc.VectorSubcoreMesh(
            core_axis_name="c", subcore_axis_name="s",
            num_cores=NC, num_subcores=NS),
        compiler_params=pltpu.CompilerParams(use_tc_tiling_on_sc=False),
        scratch_types=[
            pltpu.VMEM((N,), jnp.int32),
            pltpu.VMEM((W,), jnp.int32),
            pltpu.VMEM((W, D), jnp.float32),
            pltpu.VMEM((W, D), jnp.float32),
            pltpu.SemaphoreType.DMA,
        ],
    )


def kernel(k, v, scores, n_evict):
    del n_evict
    ss, sidx = _sort_scores(scores)
    kf = k.reshape(B * H * N, D)
    vf = v.reshape(B * H * N, D)
    kk, kv, ek, ev = _make_sc_gather()(kf, vf, sidx)
    return (
        kk.reshape(B, H, KEEP, D),
        kv.reshape(B, H, KEEP, D),
        ss[:, :KEEP],
        ek.reshape(B, H, EVICT, D),
        ev.reshape(B, H, EVICT, D),
        ss[:, KEEP:],
    )


# speedup vs baseline: 3.1098x; 1.0308x over previous
import jax
import jax.numpy as jnp
from jax import lax
from jax.experimental import pallas as pl
from jax.experimental.pallas import tpu as pltpu
from jax.experimental.pallas import tpu_sc as plsc

B, H, N, D = 8, 16, 8192, 64
KEEP = 6144
EVICT = 2048

def _sort_body(jt_ref, kt_ref, s_in_ref, ss_ref, si_ref):
    step = pl.program_id(0)

    @pl.when(step == 0)
    def _():
        ss_ref[...] = s_in_ref[...]
        si_ref[...] = lax.broadcasted_iota(jnp.int32, (B, N), 1)

    j = jt_ref[step]
    k = kt_ref[step]
    s = ss_ref[...]
    ix = si_ref[...]
    iota = lax.broadcasted_iota(jnp.int32, (B, N), 1)
    bitj = (iota & j) == 0
    sp = jnp.where(bitj, pltpu.roll(s, N - j, 1), pltpu.roll(s, j, 1))
    ip = jnp.where(bitj, pltpu.roll(ix, N - j, 1), pltpu.roll(ix, j, 1))
    up = (iota & k) == 0
    lt_peer = (sp > s) | ((sp == s) & (ip < ix))
    take = lt_peer == (bitj == up)
    ss_ref[...] = jnp.where(take, sp, s)
    si_ref[...] = jnp.where(take, ip, ix)


def _bitonic_steps():
    js, ks = [], []
    k = 2
    while k <= N:
        j = k // 2
        while j >= 1:
            js.append(j)
            ks.append(k)
            j //= 2
        k *= 2
    return js, ks


def _sort_scores(scores):
    js, ks = _bitonic_steps()
    jt = jnp.asarray(js, dtype=jnp.int32)
    kt = jnp.asarray(ks, dtype=jnp.int32)
    ss, si = pl.pallas_call(
        _sort_body,
        grid=(len(js),),
        in_specs=[
            pl.BlockSpec(memory_space=pltpu.SMEM),
            pl.BlockSpec(memory_space=pltpu.SMEM),
            pl.BlockSpec((B, N), lambda i: (0, 0)),
        ],
        out_specs=[
            pl.BlockSpec((B, N), lambda i: (0, 0)),
            pl.BlockSpec((B, N), lambda i: (0, 0)),
        ],
        out_shape=[
            jax.ShapeDtypeStruct((B, N), jnp.float32),
            jax.ShapeDtypeStruct((B, N), jnp.int32),
        ],
    )(jt, kt, scores)
    return ss, si


NC, NS = 2, 16
NW = NC * NS
TPW = (B * H) // NW
LANES = 16
W = 256
NWIN = N // W
KWIN = KEEP // W
EWIN = EVICT // W
TOTW = TPW * NWIN


def _sc_body(kf, vf, sidx_hbm,
             kk, kv, ek, ev,
             sidx_v, idxw0, idxw1, kbuf0, kbuf1, vbuf0, vbuf1,
             gsem, wsem0, wsem1):
    c = lax.axis_index("c")
    s = lax.axis_index("s")
    wid = s * NC + c
    b = wid // TPW

    pltpu.sync_copy(sidx_hbm.at[b], sidx_v)

    idxw = (idxw0, idxw1)
    kbuf = (kbuf0, kbuf1)
    vbuf = (vbuf0, vbuf1)
    wsem = (wsem0, wsem1)
    dumk = kk.at[0, 0, pl.ds(0, W)]
    dumv = kv.at[0, 0, pl.ds(0, W)]

    def half(w, p):
        t = w // NWIN
        wi = w % NWIN
        bh = wid * TPW + t
        b4 = bh // H
        h4 = bh % H

        @pl.when(w >= 2)
        def _():
            pltpu.make_async_copy(kbuf[p], dumk, wsem[p]).wait()
            pltpu.make_async_copy(vbuf[p], dumv, wsem[p]).wait()

        def mk(ci, c2):
            idxw[p][pl.ds(ci * LANES, LANES)] = (
                sidx_v[pl.ds(wi * W + ci * LANES, LANES)])
            return c2
        lax.fori_loop(0, W // LANES, mk, 0)

        ck = pltpu.async_copy(kf.at[b4, h4].at[idxw[p]], kbuf[p], gsem)
        cv = pltpu.async_copy(vf.at[b4, h4].at[idxw[p]], vbuf[p], gsem)
        ck.wait()
        cv.wait()

        @pl.when(wi < KWIN)
        def _():
            pltpu.async_copy(kbuf[p], kk.at[b4, h4, pl.ds(wi * W, W)], wsem[p])
            pltpu.async_copy(vbuf[p], kv.at[b4, h4, pl.ds(wi * W, W)], wsem[p])

        @pl.when(wi >= KWIN)
        def _():
            wj = wi - KWIN
            pltpu.async_copy(kbuf[p], ek.at[b4, h4, pl.ds(wj * W, W)], wsem[p])
            pltpu.async_copy(vbuf[p], ev.at[b4, h4, pl.ds(wj * W, W)], wsem[p])

    def outer(w2, carry):
        half(w2 * 2, 0)
        half(w2 * 2 + 1, 1)
        return carry

    lax.fori_loop(0, TOTW // 2, outer, 0)

    for p in range(2):
        pltpu.make_async_copy(kbuf[p], dumk, wsem[p]).wait()
        pltpu.make_async_copy(vbuf[p], dumv, wsem[p]).wait()


def _make_sc_gather():
    return pl.kernel(
        _sc_body,
        out_type=(
            jax.ShapeDtypeStruct((B, H, KEEP, D), jnp.float32),
            jax.ShapeDtypeStruct((B, H, KEEP, D), jnp.float32),
            jax.ShapeDtypeStruct((B, H, EVICT, D), jnp.float32),
            jax.ShapeDtypeStruct((B, H, EVICT, D), jnp.float32),
        ),
        mesh=plsc.VectorSubcoreMesh(
            core_axis_name="c", subcore_axis_name="s",
            num_cores=NC, num_subcores=NS),
        compiler_params=pltpu.CompilerParams(use_tc_tiling_on_sc=False),
        scratch_types=[
            pltpu.VMEM((N,), jnp.int32),
            pltpu.VMEM((W,), jnp.int32),
            pltpu.VMEM((W,), jnp.int32),
            pltpu.VMEM((W, D), jnp.float32),
            pltpu.VMEM((W, D), jnp.float32),
            pltpu.VMEM((W, D), jnp.float32),
            pltpu.VMEM((W, D), jnp.float32),
            pltpu.SemaphoreType.DMA,
            pltpu.SemaphoreType.DMA,
            pltpu.SemaphoreType.DMA,
        ],
    )


def kernel(k, v, scores, n_evict):
    del n_evict
    ss, sidx = _sort_scores(scores)
    kk, kv, ek, ev = _make_sc_gather()(k, v, sidx)
    return (kk, kv, ss[:, :KEEP], ek, ev, ss[:, KEEP:])
